# Initial kernel scaffold; baseline (speedup 1.0000x reference)
import jax
import jax.numpy as jnp
from jax.experimental import pallas as pl


def kernel(x, edge_index, W_lx, b_lx, alphas, betas, dxs, dys, taus, logit_scale, W_out, b_out):
    raise NotImplementedError("write your pallas kernel here")


# speedup vs baseline: 4.1556x; 4.1556x over previous
import jax
import jax.numpy as jnp
from jax import lax
from jax.experimental import pallas as pl
from jax.experimental.pallas import tpu as pltpu
from jax.experimental.pallas import tpu_sc as plsc

N = 10000
E = 320000
D_IN = 128
HID = 64
NC = 40
LAYERS = 15
DT = 0.1

NCORES = 2
NSUB = 16
NW = NCORES * NSUB
NP = 10240
ROWS_T = NP // NW
ROWS_S = NP // NSUB
CHUNK = 128
GRP = 8
ECH = 80
NGRP = ECH // GRP
EP = NW * ECH * CHUNK
F = 2 * HID
DUMMY_ROW = N

_f32 = jnp.float32
_i32 = jnp.int32


def _ks_body(q_hbm, s2_hbm, d2_hbm, z_hbm, part_hbm,
             sbuf, dbuf, r0, r1, acc, sg0, sg1):
  c = lax.axis_index("c")
  s = lax.axis_index("s")
  w = c * NSUB + s
  pltpu.sync_copy(z_hbm.at[0, pl.ds(s * ROWS_S, ROWS_S)],
                  acc.at[pl.ds(s * ROWS_S, ROWS_S)])
  plsc.subcore_barrier()
  rows = (r0, r1)
  sems = (sg0, sg1)
  for g in range(NGRP):
    pltpu.sync_copy(s2_hbm.at[pl.ds(w * ECH + g * GRP, GRP)], sbuf)
    pltpu.sync_copy(d2_hbm.at[pl.ds(w * ECH + g * GRP, GRP)], dbuf)
    desc = pltpu.async_copy(q_hbm.at[sbuf.at[0]], r0, sg0)
    for j in range(GRP):
      cur = rows[j % 2]
      desc.wait()
      if j + 1 < GRP:
        desc = pltpu.async_copy(q_hbm.at[sbuf.at[j + 1]],
                                rows[(j + 1) % 2], sems[(j + 1) % 2])
      pltpu.sync_copy(cur, acc.at[dbuf.at[j]], add=True)
  plsc.subcore_barrier()
  pltpu.sync_copy(acc.at[pl.ds(s * ROWS_S, ROWS_S)],
                  part_hbm.at[c, pl.ds(s * ROWS_S, ROWS_S)])


def _kr1_body(part_hbm, b_hbm, dis_hbm, c_hbm, q_hbm,
              p0b, p1b, bb, db, cb, qb):
  c = lax.axis_index("c")
  s = lax.axis_index("s")
  base = (c * NSUB + s) * ROWS_T
  pltpu.sync_copy(c_hbm, cb)
  k1x, mx, k1y, my = cb[3], cb[4], cb[5], cb[6]
  for i in range(ROWS_T // 64):
    st = base + i * 64
    pltpu.sync_copy(part_hbm.at[0, pl.ds(st, 64)], p0b)
    pltpu.sync_copy(part_hbm.at[1, pl.ds(st, 64)], p1b)
    pltpu.sync_copy(b_hbm.at[pl.ds(st, 64)], bb)
    pltpu.sync_copy(dis_hbm.at[pl.ds(st, 64)], db)

    def row(r, _):
      d = db[r]
      dd = d * d
      for cc in range(8):
        sl = pl.ds(cc * 16, 16)
        sv = p0b[r, sl] + p1b[r, sl]
        k1 = k1x if cc < 4 else k1y
        m = mx if cc < 4 else my
        qb[r, sl] = k1 * d * bb[r, sl] + m * dd * sv
      return 0

    lax.fori_loop(0, 64, row, 0)
    pltpu.sync_copy(qb, q_hbm.at[pl.ds(st, 64)])


def _kr2_body(part_hbm, b_hbm, w_hbm, dis_hbm, c_hbm,
              wo_hbm, bo_hbm, qo_hbm,
              p0b, p1b, bb, wb, db, cb, wob, bob, qob):
  c = lax.axis_index("c")
  s = lax.axis_index("s")
  base = (c * NSUB + s) * ROWS_T
  pltpu.sync_copy(c_hbm, cb)
  k1x, mx, k1y, my, t = cb[3], cb[4], cb[5], cb[6], cb[7]
  u1, u2, u3 = cb[8], cb[9], cb[10]
  one_t = 1.0 - t
  for i in range(ROWS_T // 64):
    st = base + i * 64
    pltpu.sync_copy(part_hbm.at[0, pl.ds(st, 64)], p0b)
    pltpu.sync_copy(part_hbm.at[1, pl.ds(st, 64)], p1b)
    pltpu.sync_copy(b_hbm.at[pl.ds(st, 64)], bb)
    pltpu.sync_copy(w_hbm.at[pl.ds(st, 64)], wb)
    pltpu.sync_copy(dis_hbm.at[pl.ds(st, 64)], db)

    def row(r, _):
      d = db[r]
      wv = []
      for cc in range(8):
        sl = pl.ds(cc * 16, 16)
        sv = p0b[r, sl] + p1b[r, sl]
        k1 = k1x if cc < 4 else k1y
        m = mx if cc < 4 else my
        xn = k1 * bb[r, sl] + m * d * sv
        wnew = one_t * wb[r, sl] + t * xn
        wob[r, sl] = wnew
        wv.append(wnew)
      for cc in range(4):
        slx = pl.ds(cc * 16, 16)
        sly = pl.ds(HID + cc * 16, 16)
        xy = wv[cc] * wv[cc + 4]
        bx = u1 * wv[cc] - u2 * xy
        by = u3 * wv[cc + 4] + u2 * xy
        bob[r, slx] = bx
        bob[r, sly] = by
        qob[r, slx] = d * bx
        qob[r, sly] = d * by
      return 0

    lax.fori_loop(0, 64, row, 0)
    pltpu.sync_copy(wob, wo_hbm.at[pl.ds(st, 64)])
    pltpu.sync_copy(bob, bo_hbm.at[pl.ds(st, 64)])
    pltpu.sync_copy(qob, qo_hbm.at[pl.ds(st, 64)])


def _kdis_body(part_hbm, dis_hbm, p0b, p1b, ob):
  c = lax.axis_index("c")
  s = lax.axis_index("s")
  base = (c * NSUB + s) * ROWS_T
  for i in range(ROWS_T // 64):
    st = base + i * 64
    pltpu.sync_copy(part_hbm.at[0, pl.ds(st, 64)], p0b)
    pltpu.sync_copy(part_hbm.at[1, pl.ds(st, 64)], p1b)

    def row(r, _):
      deg = p0b[r, pl.ds(0, 16)] + p1b[r, pl.ds(0, 16)]
      dm = jnp.maximum(deg, 1.0)
      ii = lax.bitcast_convert_type(dm, _i32)
      ii = 1597463007 - (ii >> 1)
      y = lax.bitcast_convert_type(ii, _f32)
      for _ in range(3):
        y = y * (1.5 - 0.5 * dm * y * y)
      ob[r] = jnp.where(deg > 0.5, y, 0.0)
      return 0

    lax.fori_loop(0, 64, row, 0)
    pltpu.sync_copy(ob, dis_hbm.at[pl.ds(st, 64)])


def _lift_body(x_ref, wl_ref, bl_ref, o_ref):
  h = jnp.tanh(
      lax.dot_general(x_ref[...], wl_ref[...], (((1,), (1,)), ((), ())),
                      preferred_element_type=_f32) + bl_ref[...])
  o_ref[...] = jnp.concatenate([h, jnp.ones_like(h)], axis=-1)


def _head_body(w_ref, wo_ref, b_ref, o_ref):
  xv = w_ref[...][:, :HID]
  res = lax.dot_general(xv, wo_ref[...], (((1,), (1,)), ((), ())),
                        preferred_element_type=_f32) + b_ref[...]
  o_ref[...] = res[:N, :]


def kernel(x, edge_index, W_lx, b_lx, alphas, betas, dxs, dys, taus,
           logit_scale, W_out, b_out):
  src = edge_index[0]
  dst = edge_index[1]
  pad = EP - E
  srcp = jnp.concatenate([src, jnp.zeros((pad,), _i32)])
  dstp = jnp.concatenate([dst, jnp.full((pad,), DUMMY_ROW, _i32)])
  src2d = srcp.reshape(NW * ECH, CHUNK)
  dst2d = dstp.reshape(NW * ECH, CHUNK)
  xp = jnp.pad(x, ((0, NP - N), (0, 0)))
  zeros_pp = jnp.zeros((2, NP, F), _f32)

  t = jax.nn.sigmoid(taus).astype(_f32)
  u1 = 1.0 + DT * alphas
  u2 = DT * betas
  u3 = 1.0 - DT * alphas
  k1x = 1.0 / (1.0 + DT * dxs)
  mx = DT * dxs * k1x
  k1y = 1.0 / (1.0 + DT * dys)
  my = DT * dys * k1y
  z = jnp.zeros((LAYERS,), _f32)
  cols = jnp.stack([u1, u2, u3, k1x, mx, k1y, my, t,
                    jnp.roll(u1, -1), jnp.roll(u2, -1), jnp.roll(u3, -1),
                    z, z, z, z, z], axis=1)
  consts = jnp.broadcast_to(cols[:, :, None], (LAYERS, 16, 16)).astype(_f32)
  ca0_row = (jnp.zeros((16,), _f32)
             .at[8].set(u1[0]).at[9].set(u2[0]).at[10].set(u3[0]))
  ca0 = jnp.broadcast_to(ca0_row[:, None], (16, 16)).astype(_f32)

  mesh = plsc.VectorSubcoreMesh(core_axis_name="c", subcore_axis_name="s",
                                num_cores=NCORES, num_subcores=NSUB)
  sds = jax.ShapeDtypeStruct

  k_s = pl.kernel(
      _ks_body,
      out_type=sds((2, NP, F), _f32),
      mesh=mesh,
      scratch_types=[
          pltpu.VMEM((GRP, CHUNK), _i32),
          pltpu.VMEM((GRP, CHUNK), _i32),
          pltpu.VMEM((CHUNK, F), _f32),
          pltpu.VMEM((CHUNK, F), _f32),
          pltpu.VMEM_SHARED((NP, F), _f32),
          pltpu.SemaphoreType.DMA,
          pltpu.SemaphoreType.DMA,
      ],
      name="ppgnn_edge_pass",
  )
  k_r1 = pl.kernel(
      _kr1_body,
      out_type=sds((NP, F), _f32),
      mesh=mesh,
      scratch_types=[
          pltpu.VMEM((64, F), _f32),
          pltpu.VMEM((64, F), _f32),
          pltpu.VMEM((64, F), _f32),
          pltpu.VMEM((64, 16), _f32),
          pltpu.VMEM((16, 16), _f32),
          pltpu.VMEM((64, F), _f32),
      ],
      name="ppgnn_jacobi1",
  )
  k_r2 = pl.kernel(
      _kr2_body,
      out_type=(sds((NP, F), _f32), sds((NP, F), _f32), sds((NP, F), _f32)),
      mesh=mesh,
      scratch_types=[
          pltpu.VMEM((64, F), _f32),
          pltpu.VMEM((64, F), _f32),
          pltpu.VMEM((64, F), _f32),
          pltpu.VMEM((64, F), _f32),
          pltpu.VMEM((64, 16), _f32),
          pltpu.VMEM((16, 16), _f32),
          pltpu.VMEM((64, F), _f32),
          pltpu.VMEM((64, F), _f32),
          pltpu.VMEM((64, F), _f32),
      ],
      name="ppgnn_jacobi2_react",
  )
  k_dis = pl.kernel(
      _kdis_body,
      out_type=sds((NP, 16), _f32),
      mesh=mesh,
      scratch_types=[
          pltpu.VMEM((64, F), _f32),
          pltpu.VMEM((64, F), _f32),
          pltpu.VMEM((64, 16), _f32),
      ],
      name="ppgnn_dis",
  )

  w0 = pl.pallas_call(
      _lift_body,
      out_shape=sds((NP, F), _f32),
  )(xp, W_lx, b_lx.reshape(1, HID))

  ones_pp = jnp.ones((NP, F), _f32)
  part_deg = k_s(ones_pp, src2d, dst2d, zeros_pp)
  dis16 = k_dis(part_deg)

  w1, b0, q0 = k_r2(zeros_pp, w0, w0, dis16, ca0)

  def layer_step(carry, cl):
    w, b, q = carry
    part = k_s(q, src2d, dst2d, zeros_pp)
    q1 = k_r1(part, b, dis16, cl)
    part2 = k_s(q1, src2d, dst2d, zeros_pp)
    w2, b2, q2 = k_r2(part2, b, w, dis16, cl)
    return (w2, b2, q2), None

  (w_fin, _, _), _ = lax.scan(layer_step, (w1, b0, q0), consts)

  out = pl.pallas_call(
      _head_body,
      out_shape=sds((N, NC), _f32),
  )(w_fin, logit_scale.astype(_f32) * W_out, b_out.reshape(1, NC))
  return out


# speedup vs baseline: 4.2139x; 1.0140x over previous
import jax
import jax.numpy as jnp
from jax import lax
from jax.experimental import pallas as pl
from jax.experimental.pallas import tpu as pltpu
from jax.experimental.pallas import tpu_sc as plsc

N = 10000
E = 320000
D_IN = 128
HID = 64
NC = 40
LAYERS = 15
DT = 0.1

NCORES = 2
NSUB = 16
NW = NCORES * NSUB
NP = 10240
ROWS_T = NP // NW
ROWS_S = NP // NSUB
CHUNK = 128
GRP = 8
ECH = 80
NGRP = ECH // GRP
EP = NW * ECH * CHUNK
F = 2 * HID
DUMMY_ROW = N

_f32 = jnp.float32
_i32 = jnp.int32


def _ks_body(q_hbm, s2_hbm, d2_hbm, z_hbm, part_hbm,
             sbufa, sbufb, dbufa, dbufb, r0, r1,
             acc, sg0, sg1, ss0, ss1):
  c = lax.axis_index("c")
  s = lax.axis_index("s")
  w = c * NSUB + s
  pltpu.sync_copy(z_hbm.at[0, pl.ds(s * ROWS_S, ROWS_S)],
                  acc.at[pl.ds(s * ROWS_S, ROWS_S)])
  plsc.subcore_barrier()
  rows = (r0, r1)
  gsems = (sg0, sg1)
  ssems = (ss0, ss1)
  sbufs = (sbufa, sbufb)
  dbufs = (dbufa, dbufb)
  pltpu.sync_copy(s2_hbm.at[pl.ds(w * ECH, GRP)], sbufa)
  pltpu.sync_copy(d2_hbm.at[pl.ds(w * ECH, GRP)], dbufa)
  gdesc = pltpu.async_copy(q_hbm.at[sbufa.at[0]], r0, sg0)
  sdesc = None
  for g in range(NGRP):
    gp = g % 2
    if g + 1 < NGRP:
      pltpu.sync_copy(s2_hbm.at[pl.ds(w * ECH + (g + 1) * GRP, GRP)],
                      sbufs[1 - gp])
    for j in range(GRP):
      ci = g * GRP + j
      gdesc.wait()
      if sdesc is not None:
        sdesc.wait()
      if j == 0 and g + 1 < NGRP:
        pltpu.sync_copy(d2_hbm.at[pl.ds(w * ECH + (g + 1) * GRP, GRP)],
                        dbufs[1 - gp])
      if ci + 1 < ECH:
        nsb = sbufs[gp] if j + 1 < GRP else sbufs[1 - gp]
        gdesc = pltpu.async_copy(q_hbm.at[nsb.at[(j + 1) % GRP]],
                                 rows[(ci + 1) % 2], gsems[(ci + 1) % 2])
      sdesc = pltpu.async_copy(rows[ci % 2], acc.at[dbufs[gp].at[j]],
                               ssems[ci % 2], add=True)
  sdesc.wait()
  plsc.subcore_barrier()
  pltpu.sync_copy(acc.at[pl.ds(s * ROWS_S, ROWS_S)],
                  part_hbm.at[c, pl.ds(s * ROWS_S, ROWS_S)])


def _kr1_body(part_hbm, b_hbm, dis_hbm, c_hbm, q_hbm,
              p0b, p1b, bb, db, cb, qb):
  c = lax.axis_index("c")
  s = lax.axis_index("s")
  base = (c * NSUB + s) * ROWS_T
  pltpu.sync_copy(c_hbm, cb)
  k1x, mx, k1y, my = cb[3], cb[4], cb[5], cb[6]
  for i in range(ROWS_T // 64):
    st = base + i * 64
    pltpu.sync_copy(part_hbm.at[0, pl.ds(st, 64)], p0b)
    pltpu.sync_copy(part_hbm.at[1, pl.ds(st, 64)], p1b)
    pltpu.sync_copy(b_hbm.at[pl.ds(st, 64)], bb)
    pltpu.sync_copy(dis_hbm.at[pl.ds(st, 64)], db)

    def row(r, _):
      d = db[r]
      dd = d * d
      for cc in range(8):
        sl = pl.ds(cc * 16, 16)
        sv = p0b[r, sl] + p1b[r, sl]
        k1 = k1x if cc < 4 else k1y
        m = mx if cc < 4 else my
        qb[r, sl] = k1 * d * bb[r, sl] + m * dd * sv
      return 0

    lax.fori_loop(0, 64, row, 0)
    pltpu.sync_copy(qb, q_hbm.at[pl.ds(st, 64)])


def _kr2_body(part_hbm, b_hbm, w_hbm, dis_hbm, c_hbm,
              wo_hbm, bo_hbm, qo_hbm,
              p0b, p1b, bb, wb, db, cb, wob, bob, qob):
  c = lax.axis_index("c")
  s = lax.axis_index("s")
  base = (c * NSUB + s) * ROWS_T
  pltpu.sync_copy(c_hbm, cb)
  k1x, mx, k1y, my, t = cb[3], cb[4], cb[5], cb[6], cb[7]
  u1, u2, u3 = cb[8], cb[9], cb[10]
  one_t = 1.0 - t
  for i in range(ROWS_T // 64):
    st = base + i * 64
    pltpu.sync_copy(part_hbm.at[0, pl.ds(st, 64)], p0b)
    pltpu.sync_copy(part_hbm.at[1, pl.ds(st, 64)], p1b)
    pltpu.sync_copy(b_hbm.at[pl.ds(st, 64)], bb)
    pltpu.sync_copy(w_hbm.at[pl.ds(st, 64)], wb)
    pltpu.sync_copy(dis_hbm.at[pl.ds(st, 64)], db)

    def row(r, _):
      d = db[r]
      wv = []
      for cc in range(8):
        sl = pl.ds(cc * 16, 16)
        sv = p0b[r, sl] + p1b[r, sl]
        k1 = k1x if cc < 4 else k1y
        m = mx if cc < 4 else my
        xn = k1 * bb[r, sl] + m * d * sv
        wnew = one_t * wb[r, sl] + t * xn
        wob[r, sl] = wnew
        wv.append(wnew)
      for cc in range(4):
        slx = pl.ds(cc * 16, 16)
        sly = pl.ds(HID + cc * 16, 16)
        xy = wv[cc] * wv[cc + 4]
        bx = u1 * wv[cc] - u2 * xy
        by = u3 * wv[cc + 4] + u2 * xy
        bob[r, slx] = bx
        bob[r, sly] = by
        qob[r, slx] = d * bx
        qob[r, sly] = d * by
      return 0

    lax.fori_loop(0, 64, row, 0)
    pltpu.sync_copy(wob, wo_hbm.at[pl.ds(st, 64)])
    pltpu.sync_copy(bob, bo_hbm.at[pl.ds(st, 64)])
    pltpu.sync_copy(qob, qo_hbm.at[pl.ds(st, 64)])


def _kdis_body(part_hbm, dis_hbm, p0b, p1b, ob):
  c = lax.axis_index("c")
  s = lax.axis_index("s")
  base = (c * NSUB + s) * ROWS_T
  for i in range(ROWS_T // 64):
    st = base + i * 64
    pltpu.sync_copy(part_hbm.at[0, pl.ds(st, 64)], p0b)
    pltpu.sync_copy(part_hbm.at[1, pl.ds(st, 64)], p1b)

    def row(r, _):
      deg = p0b[r, pl.ds(0, 16)] + p1b[r, pl.ds(0, 16)]
      dm = jnp.maximum(deg, 1.0)
      ii = lax.bitcast_convert_type(dm, _i32)
      ii = 1597463007 - (ii >> 1)
      y = lax.bitcast_convert_type(ii, _f32)
      for _ in range(3):
        y = y * (1.5 - 0.5 * dm * y * y)
      ob[r] = jnp.where(deg > 0.5, y, 0.0)
      return 0

    lax.fori_loop(0, 64, row, 0)
    pltpu.sync_copy(ob, dis_hbm.at[pl.ds(st, 64)])


def _lift_body(x_ref, wl_ref, bl_ref, o_ref):
  h = jnp.tanh(
      lax.dot_general(x_ref[...], wl_ref[...], (((1,), (1,)), ((), ())),
                      preferred_element_type=_f32) + bl_ref[...])
  o_ref[...] = jnp.concatenate([h, jnp.ones_like(h)], axis=-1)


def _head_body(w_ref, wo_ref, b_ref, o_ref):
  xv = w_ref[...][:, :HID]
  res = lax.dot_general(xv, wo_ref[...], (((1,), (1,)), ((), ())),
                        preferred_element_type=_f32) + b_ref[...]
  o_ref[...] = res[:N, :]


def kernel(x, edge_index, W_lx, b_lx, alphas, betas, dxs, dys, taus,
           logit_scale, W_out, b_out):
  src = edge_index[0]
  dst = edge_index[1]
  pad = EP - E
  srcp = jnp.concatenate([src, jnp.zeros((pad,), _i32)])
  dstp = jnp.concatenate([dst, jnp.full((pad,), DUMMY_ROW, _i32)])
  src2d = srcp.reshape(NW * ECH, CHUNK)
  dst2d = dstp.reshape(NW * ECH, CHUNK)
  xp = jnp.pad(x, ((0, NP - N), (0, 0)))
  zeros_pp = jnp.zeros((2, NP, F), _f32)

  t = jax.nn.sigmoid(taus).astype(_f32)
  u1 = 1.0 + DT * alphas
  u2 = DT * betas
  u3 = 1.0 - DT * alphas
  k1x = 1.0 / (1.0 + DT * dxs)
  mx = DT * dxs * k1x
  k1y = 1.0 / (1.0 + DT * dys)
  my = DT * dys * k1y
  z = jnp.zeros((LAYERS,), _f32)
  cols = jnp.stack([u1, u2, u3, k1x, mx, k1y, my, t,
                    jnp.roll(u1, -1), jnp.roll(u2, -1), jnp.roll(u3, -1),
                    z, z, z, z, z], axis=1)
  consts = jnp.broadcast_to(cols[:, :, None], (LAYERS, 16, 16)).astype(_f32)
  ca0_row = (jnp.zeros((16,), _f32)
             .at[8].set(u1[0]).at[9].set(u2[0]).at[10].set(u3[0]))
  ca0 = jnp.broadcast_to(ca0_row[:, None], (16, 16)).astype(_f32)

  mesh = plsc.VectorSubcoreMesh(core_axis_name="c", subcore_axis_name="s",
                                num_cores=NCORES, num_subcores=NSUB)
  sds = jax.ShapeDtypeStruct

  k_s = pl.kernel(
      _ks_body,
      out_type=sds((2, NP, F), _f32),
      mesh=mesh,
      scratch_types=[
          pltpu.VMEM((GRP, CHUNK), _i32),
          pltpu.VMEM((GRP, CHUNK), _i32),
          pltpu.VMEM((GRP, CHUNK), _i32),
          pltpu.VMEM((GRP, CHUNK), _i32),
          pltpu.VMEM((CHUNK, F), _f32),
          pltpu.VMEM((CHUNK, F), _f32),
          pltpu.VMEM_SHARED((NP, F), _f32),
          pltpu.SemaphoreType.DMA,
          pltpu.SemaphoreType.DMA,
          pltpu.SemaphoreType.DMA,
          pltpu.SemaphoreType.DMA,
      ],
      name="ppgnn_edge_pass",
  )
  k_r1 = pl.kernel(
      _kr1_body,
      out_type=sds((NP, F), _f32),
      mesh=mesh,
      scratch_types=[
          pltpu.VMEM((64, F), _f32),
          pltpu.VMEM((64, F), _f32),
          pltpu.VMEM((64, F), _f32),
          pltpu.VMEM((64, 16), _f32),
          pltpu.VMEM((16, 16), _f32),
          pltpu.VMEM((64, F), _f32),
      ],
      name="ppgnn_jacobi1",
  )
  k_r2 = pl.kernel(
      _kr2_body,
      out_type=(sds((NP, F), _f32), sds((NP, F), _f32), sds((NP, F), _f32)),
      mesh=mesh,
      scratch_types=[
          pltpu.VMEM((64, F), _f32),
          pltpu.VMEM((64, F), _f32),
          pltpu.VMEM((64, F), _f32),
          pltpu.VMEM((64, F), _f32),
          pltpu.VMEM((64, 16), _f32),
          pltpu.VMEM((16, 16), _f32),
          pltpu.VMEM((64, F), _f32),
          pltpu.VMEM((64, F), _f32),
          pltpu.VMEM((64, F), _f32),
      ],
      name="ppgnn_jacobi2_react",
  )
  k_dis = pl.kernel(
      _kdis_body,
      out_type=sds((NP, 16), _f32),
      mesh=mesh,
      scratch_types=[
          pltpu.VMEM((64, F), _f32),
          pltpu.VMEM((64, F), _f32),
          pltpu.VMEM((64, 16), _f32),
      ],
      name="ppgnn_dis",
  )

  w0 = pl.pallas_call(
      _lift_body,
      out_shape=sds((NP, F), _f32),
  )(xp, W_lx, b_lx.reshape(1, HID))

  ones_pp = jnp.ones((NP, F), _f32)
  part_deg = k_s(ones_pp, src2d, dst2d, zeros_pp)
  dis16 = k_dis(part_deg)

  w1, b0, q0 = k_r2(zeros_pp, w0, w0, dis16, ca0)

  def layer_step(carry, cl):
    w, b, q = carry
    part = k_s(q, src2d, dst2d, zeros_pp)
    q1 = k_r1(part, b, dis16, cl)
    part2 = k_s(q1, src2d, dst2d, zeros_pp)
    w2, b2, q2 = k_r2(part2, b, w, dis16, cl)
    return (w2, b2, q2), None

  (w_fin, _, _), _ = lax.scan(layer_step, (w1, b0, q0), consts)

  out = pl.pallas_call(
      _head_body,
      out_shape=sds((N, NC), _f32),
  )(w_fin, logit_scale.astype(_f32) * W_out, b_out.reshape(1, NC))
  return out


# speedup vs baseline: 4.2860x; 1.0171x over previous
import jax
import jax.numpy as jnp
from jax import lax
from jax.experimental import pallas as pl
from jax.experimental.pallas import tpu as pltpu
from jax.experimental.pallas import tpu_sc as plsc

N = 10000
E = 320000
D_IN = 128
HID = 64
NC = 40
LAYERS = 15
DT = 0.1

NCORES = 2
NSUB = 16
NW = NCORES * NSUB
NP = 10240
ROWS_T = NP // NW
ROWS_S = NP // NSUB
CHUNK = 128
GRP = 8
ECH = 80
NGRP = ECH // GRP
EP = NW * ECH * CHUNK
F = 2 * HID
DUMMY_ROW = N

_f32 = jnp.float32
_i32 = jnp.int32


def _ks_body(q_hbm, s2_hbm, d2_hbm, z_hbm, part_hbm,
             sbufa, sbufb, dbufa, dbufb, r0, r1,
             acc, sg0, sg1, ss0, ss1):
  c = lax.axis_index("c")
  s = lax.axis_index("s")
  w = c * NSUB + s
  pltpu.sync_copy(z_hbm.at[0, pl.ds(s * ROWS_S, ROWS_S)],
                  acc.at[pl.ds(s * ROWS_S, ROWS_S)])
  plsc.subcore_barrier()
  rows = (r0, r1)
  gsems = (sg0, sg1)
  ssems = (ss0, ss1)
  sbufs = (sbufa, sbufb)
  dbufs = (dbufa, dbufb)
  pltpu.sync_copy(s2_hbm.at[pl.ds(w * ECH, GRP)], sbufa)
  pltpu.sync_copy(d2_hbm.at[pl.ds(w * ECH, GRP)], dbufa)
  gdesc = pltpu.async_copy(q_hbm.at[sbufa.at[0]], r0, sg0)
  sdesc = None
  for g in range(NGRP):
    gp = g % 2
    if g + 1 < NGRP:
      pltpu.sync_copy(s2_hbm.at[pl.ds(w * ECH + (g + 1) * GRP, GRP)],
                      sbufs[1 - gp])
    for j in range(GRP):
      ci = g * GRP + j
      gdesc.wait()
      if sdesc is not None:
        sdesc.wait()
      if j == 0 and g + 1 < NGRP:
        pltpu.sync_copy(d2_hbm.at[pl.ds(w * ECH + (g + 1) * GRP, GRP)],
                        dbufs[1 - gp])
      if ci + 1 < ECH:
        nsb = sbufs[gp] if j + 1 < GRP else sbufs[1 - gp]
        gdesc = pltpu.async_copy(q_hbm.at[nsb.at[(j + 1) % GRP]],
                                 rows[(ci + 1) % 2], gsems[(ci + 1) % 2])
      sdesc = pltpu.async_copy(rows[ci % 2], acc.at[dbufs[gp].at[j]],
                               ssems[ci % 2], add=True)
  sdesc.wait()
  plsc.subcore_barrier()
  pltpu.sync_copy(acc.at[pl.ds(s * ROWS_S, ROWS_S)],
                  part_hbm.at[c, pl.ds(s * ROWS_S, ROWS_S)])


def _kr1_body(part_hbm, b_hbm, dis_hbm, c_hbm, q_hbm,
              p0b, p1b, bb, db, cb, qb):
  c = lax.axis_index("c")
  s = lax.axis_index("s")
  base = (c * NSUB + s) * ROWS_T
  pltpu.sync_copy(c_hbm, cb)
  k1x, mx, k1y, my = cb[3], cb[4], cb[5], cb[6]
  for i in range(ROWS_T // 64):
    st = base + i * 64
    pltpu.sync_copy(part_hbm.at[0, pl.ds(st, 64)], p0b)
    pltpu.sync_copy(part_hbm.at[1, pl.ds(st, 64)], p1b)
    pltpu.sync_copy(b_hbm.at[pl.ds(st, 64)], bb)
    pltpu.sync_copy(dis_hbm.at[pl.ds(st, 64)], db)

    def row(r, _):
      d = db[r]
      dd = d * d
      for cc in range(8):
        sl = pl.ds(cc * 16, 16)
        sv = p0b[r, sl] + p1b[r, sl]
        k1 = k1x if cc < 4 else k1y
        m = mx if cc < 4 else my
        qb[r, sl] = k1 * d * bb[r, sl] + m * dd * sv
      return 0

    lax.fori_loop(0, 64, row, 0)
    pltpu.sync_copy(qb, q_hbm.at[pl.ds(st, 64)])


def _kr2_body(part_hbm, b_hbm, w_hbm, dis_hbm, c_hbm,
              wo_hbm, bo_hbm, qo_hbm,
              p0b, p1b, bb, wb, db, cb, wob, bob, qob):
  c = lax.axis_index("c")
  s = lax.axis_index("s")
  base = (c * NSUB + s) * ROWS_T
  pltpu.sync_copy(c_hbm, cb)
  k1x, mx, k1y, my, t = cb[3], cb[4], cb[5], cb[6], cb[7]
  u1, u2, u3 = cb[8], cb[9], cb[10]
  one_t = 1.0 - t
  for i in range(ROWS_T // 64):
    st = base + i * 64
    pltpu.sync_copy(part_hbm.at[0, pl.ds(st, 64)], p0b)
    pltpu.sync_copy(part_hbm.at[1, pl.ds(st, 64)], p1b)
    pltpu.sync_copy(b_hbm.at[pl.ds(st, 64)], bb)
    pltpu.sync_copy(w_hbm.at[pl.ds(st, 64)], wb)
    pltpu.sync_copy(dis_hbm.at[pl.ds(st, 64)], db)

    def row(r, _):
      d = db[r]
      wv = []
      for cc in range(8):
        sl = pl.ds(cc * 16, 16)
        sv = p0b[r, sl] + p1b[r, sl]
        k1 = k1x if cc < 4 else k1y
        m = mx if cc < 4 else my
        xn = k1 * bb[r, sl] + m * d * sv
        wnew = one_t * wb[r, sl] + t * xn
        wob[r, sl] = wnew
        wv.append(wnew)
      for cc in range(4):
        slx = pl.ds(cc * 16, 16)
        sly = pl.ds(HID + cc * 16, 16)
        xy = wv[cc] * wv[cc + 4]
        bx = u1 * wv[cc] - u2 * xy
        by = u3 * wv[cc + 4] + u2 * xy
        bob[r, slx] = bx
        bob[r, sly] = by
        qob[r, slx] = d * bx
        qob[r, sly] = d * by
      return 0

    lax.fori_loop(0, 64, row, 0)
    pltpu.sync_copy(wob, wo_hbm.at[pl.ds(st, 64)])
    pltpu.sync_copy(bob, bo_hbm.at[pl.ds(st, 64)])
    pltpu.sync_copy(qob, qo_hbm.at[pl.ds(st, 64)])


def _kdis_body(part_hbm, dis_hbm, p0b, p1b, ob):
  c = lax.axis_index("c")
  s = lax.axis_index("s")
  base = (c * NSUB + s) * ROWS_T
  for i in range(ROWS_T // 64):
    st = base + i * 64
    pltpu.sync_copy(part_hbm.at[0, pl.ds(st, 64)], p0b)
    pltpu.sync_copy(part_hbm.at[1, pl.ds(st, 64)], p1b)

    def row(r, _):
      deg = p0b[r, pl.ds(0, 16)] + p1b[r, pl.ds(0, 16)]
      dm = jnp.maximum(deg, 1.0)
      ii = lax.bitcast_convert_type(dm, _i32)
      ii = 1597463007 - (ii >> 1)
      y = lax.bitcast_convert_type(ii, _f32)
      for _ in range(3):
        y = y * (1.5 - 0.5 * dm * y * y)
      ob[r] = jnp.where(deg > 0.5, y, 0.0)
      return 0

    lax.fori_loop(0, 64, row, 0)
    pltpu.sync_copy(ob, dis_hbm.at[pl.ds(st, 64)])


def _lift_body(x_ref, wl_ref, bl_ref, o_ref):
  h = jnp.tanh(
      lax.dot_general(x_ref[...], wl_ref[...], (((1,), (1,)), ((), ())),
                      preferred_element_type=_f32) + bl_ref[...])
  o_ref[...] = jnp.concatenate([h, jnp.ones_like(h)], axis=-1)


def _head_body(w_ref, wo_ref, b_ref, o_ref):
  xv = w_ref[...][:, :HID]
  res = lax.dot_general(xv, wo_ref[...], (((1,), (1,)), ((), ())),
                        preferred_element_type=_f32) + b_ref[...]
  o_ref[...] = res[:N, :]


def kernel(x, edge_index, W_lx, b_lx, alphas, betas, dxs, dys, taus,
           logit_scale, W_out, b_out):
  src = edge_index[0]
  dst = edge_index[1]
  pad = EP - E
  srcp = jnp.concatenate([src, jnp.zeros((pad,), _i32)])
  dstp = jnp.concatenate([dst, jnp.full((pad,), DUMMY_ROW, _i32)])
  order = jnp.argsort(dstp)
  srcp = srcp[order]
  dstp = dstp[order]
  src2d = srcp.reshape(NW * ECH, CHUNK)
  dst2d = dstp.reshape(NW * ECH, CHUNK)
  xp = jnp.pad(x, ((0, NP - N), (0, 0)))
  zeros_pp = jnp.zeros((2, NP, F), _f32)

  t = jax.nn.sigmoid(taus).astype(_f32)
  u1 = 1.0 + DT * alphas
  u2 = DT * betas
  u3 = 1.0 - DT * alphas
  k1x = 1.0 / (1.0 + DT * dxs)
  mx = DT * dxs * k1x
  k1y = 1.0 / (1.0 + DT * dys)
  my = DT * dys * k1y
  z = jnp.zeros((LAYERS,), _f32)
  cols = jnp.stack([u1, u2, u3, k1x, mx, k1y, my, t,
                    jnp.roll(u1, -1), jnp.roll(u2, -1), jnp.roll(u3, -1),
                    z, z, z, z, z], axis=1)
  consts = jnp.broadcast_to(cols[:, :, None], (LAYERS, 16, 16)).astype(_f32)
  ca0_row = (jnp.zeros((16,), _f32)
             .at[8].set(u1[0]).at[9].set(u2[0]).at[10].set(u3[0]))
  ca0 = jnp.broadcast_to(ca0_row[:, None], (16, 16)).astype(_f32)

  mesh = plsc.VectorSubcoreMesh(core_axis_name="c", subcore_axis_name="s",
                                num_cores=NCORES, num_subcores=NSUB)
  sds = jax.ShapeDtypeStruct

  k_s = pl.kernel(
      _ks_body,
      out_type=sds((2, NP, F), _f32),
      mesh=mesh,
      scratch_types=[
          pltpu.VMEM((GRP, CHUNK), _i32),
          pltpu.VMEM((GRP, CHUNK), _i32),
          pltpu.VMEM((GRP, CHUNK), _i32),
          pltpu.VMEM((GRP, CHUNK), _i32),
          pltpu.VMEM((CHUNK, F), _f32),
          pltpu.VMEM((CHUNK, F), _f32),
          pltpu.VMEM_SHARED((NP, F), _f32),
          pltpu.SemaphoreType.DMA,
          pltpu.SemaphoreType.DMA,
          pltpu.SemaphoreType.DMA,
          pltpu.SemaphoreType.DMA,
      ],
      name="ppgnn_edge_pass",
  )
  k_r1 = pl.kernel(
      _kr1_body,
      out_type=sds((NP, F), _f32),
      mesh=mesh,
      scratch_types=[
          pltpu.VMEM((64, F), _f32),
          pltpu.VMEM((64, F), _f32),
          pltpu.VMEM((64, F), _f32),
          pltpu.VMEM((64, 16), _f32),
          pltpu.VMEM((16, 16), _f32),
          pltpu.VMEM((64, F), _f32),
      ],
      name="ppgnn_jacobi1",
  )
  k_r2 = pl.kernel(
      _kr2_body,
      out_type=(sds((NP, F), _f32), sds((NP, F), _f32), sds((NP, F), _f32)),
      mesh=mesh,
      scratch_types=[
          pltpu.VMEM((64, F), _f32),
          pltpu.VMEM((64, F), _f32),
          pltpu.VMEM((64, F), _f32),
          pltpu.VMEM((64, F), _f32),
          pltpu.VMEM((64, 16), _f32),
          pltpu.VMEM((16, 16), _f32),
          pltpu.VMEM((64, F), _f32),
          pltpu.VMEM((64, F), _f32),
          pltpu.VMEM((64, F), _f32),
      ],
      name="ppgnn_jacobi2_react",
  )
  k_dis = pl.kernel(
      _kdis_body,
      out_type=sds((NP, 16), _f32),
      mesh=mesh,
      scratch_types=[
          pltpu.VMEM((64, F), _f32),
          pltpu.VMEM((64, F), _f32),
          pltpu.VMEM((64, 16), _f32),
      ],
      name="ppgnn_dis",
  )

  w0 = pl.pallas_call(
      _lift_body,
      out_shape=sds((NP, F), _f32),
  )(xp, W_lx, b_lx.reshape(1, HID))

  ones_pp = jnp.ones((NP, F), _f32)
  part_deg = k_s(ones_pp, src2d, dst2d, zeros_pp)
  dis16 = k_dis(part_deg)

  w1, b0, q0 = k_r2(zeros_pp, w0, w0, dis16, ca0)

  def layer_step(carry, cl):
    w, b, q = carry
    part = k_s(q, src2d, dst2d, zeros_pp)
    q1 = k_r1(part, b, dis16, cl)
    part2 = k_s(q1, src2d, dst2d, zeros_pp)
    w2, b2, q2 = k_r2(part2, b, w, dis16, cl)
    return (w2, b2, q2), None

  (w_fin, _, _), _ = lax.scan(layer_step, (w1, b0, q0), consts)

  out = pl.pallas_call(
      _head_body,
      out_shape=sds((N, NC), _f32),
  )(w_fin, logit_scale.astype(_f32) * W_out, b_out.reshape(1, NC))
  return out
